# baseline (speedup 1.0000x reference)
import jax
import jax.numpy as jnp
from jax.experimental import pallas as pl

_GRID = 25


def _copy_body(x_ref, xo_ref):
    xo_ref[...] = x_ref[...]


def kernel(x, edge_index, edge_attr):
    del edge_index
    n_nodes, d_feat = x.shape
    xb = n_nodes // _GRID
    x_out = pl.pallas_call(
        _copy_body,
        grid=(_GRID,),
        in_specs=[pl.BlockSpec((xb, d_feat), lambda i: (i, 0))],
        out_specs=pl.BlockSpec((xb, d_feat), lambda i: (i, 0)),
        out_shape=jax.ShapeDtypeStruct(x.shape, x.dtype),
    )(x)
    return (x_out, edge_attr)


# speedup vs baseline: 1.4338x; 1.4338x over previous
import jax
import jax.numpy as jnp
from jax.experimental import pallas as pl

_GRID = 5


def _copy_body(x_ref, xo_ref):
    xo_ref[...] = x_ref[...]


def kernel(x, edge_index, edge_attr):
    del edge_index
    n_nodes, d_feat = x.shape
    xb = n_nodes // _GRID
    x_out = pl.pallas_call(
        _copy_body,
        grid=(_GRID,),
        in_specs=[pl.BlockSpec((xb, d_feat), lambda i: (i, 0))],
        out_specs=pl.BlockSpec((xb, d_feat), lambda i: (i, 0)),
        out_shape=jax.ShapeDtypeStruct(x.shape, x.dtype),
    )(x)
    return (x_out, edge_attr)


# speedup vs baseline: 1.6082x; 1.1217x over previous
import jax
import jax.numpy as jnp
from jax.experimental import pallas as pl

_GRID = 2


def _copy_body(x_ref, xo_ref):
    xo_ref[...] = x_ref[...]


def kernel(x, edge_index, edge_attr):
    del edge_index
    n_nodes, d_feat = x.shape
    xb = n_nodes // _GRID
    x_out = pl.pallas_call(
        _copy_body,
        grid=(_GRID,),
        in_specs=[pl.BlockSpec((xb, d_feat), lambda i: (i, 0))],
        out_specs=pl.BlockSpec((xb, d_feat), lambda i: (i, 0)),
        out_shape=jax.ShapeDtypeStruct(x.shape, x.dtype),
    )(x)
    return (x_out, edge_attr)
